# baseline (speedup 1.0000x reference)
import functools

import jax
import jax.numpy as jnp
from jax.experimental import pallas as pl
from jax.experimental.pallas import tpu as pltpu

_NUM_GRAPHS = 16


def _linear_kernel(x_ref, w_ref, b_ref, o_ref):
    o_ref[...] = (
        jnp.dot(x_ref[...], w_ref[...], preferred_element_type=jnp.float32)
        + b_ref[...]
    )


def _linear(x, W, b, blk=1000):
    n, k = x.shape
    m = W.shape[1]
    return pl.pallas_call(
        _linear_kernel,
        grid=(n // blk,),
        in_specs=[
            pl.BlockSpec((blk, k), lambda i: (i, 0)),
            pl.BlockSpec((k, m), lambda i: (0, 0)),
            pl.BlockSpec((1, m), lambda i: (0, 0)),
        ],
        out_specs=pl.BlockSpec((blk, m), lambda i: (i, 0)),
        out_shape=jax.ShapeDtypeStruct((n, m), jnp.float32),
    )(x, W, b.reshape(1, m))


def _degree_kernel(dst_ref, deg_ref):
    @pl.when(pl.program_id(0) == 0)
    def _():
        deg_ref[...] = jnp.zeros_like(deg_ref)

    def body(e, carry):
        d = dst_ref[e]
        deg_ref[pl.ds(d, 1), :] = deg_ref[pl.ds(d, 1), :] + 1.0
        return carry

    jax.lax.fori_loop(0, dst_ref.shape[0], body, 0)


def _degree(dst, n, chunk=512):
    e = dst.shape[0]
    return pl.pallas_call(
        _degree_kernel,
        grid=(e // chunk,),
        in_specs=[
            pl.BlockSpec((chunk,), lambda i: (i,), memory_space=pltpu.SMEM)
        ],
        out_specs=pl.BlockSpec((n, 1), lambda i: (0, 0)),
        out_shape=jax.ShapeDtypeStruct((n, 1), jnp.float32),
    )(dst)


def _pna_scatter_kernel(src_ref, dst_ref, a_ref, b_ref, c_ref,
                        o1_ref, o2_ref, *, chunk, half, op):
    i = pl.program_id(0)

    @pl.when(i == 0)
    def _():
        if op == 'sum':
            o1_ref[...] = jnp.zeros_like(o1_ref)
            o2_ref[...] = jnp.zeros_like(o2_ref)
        else:
            o1_ref[...] = jnp.full_like(o1_ref, -jnp.inf)
            o2_ref[...] = jnp.full_like(o2_ref, jnp.inf)

    c0 = c_ref[0:1, :]
    c1 = c_ref[1:2, :]
    base = i * chunk

    def body(e, carry):
        s = src_ref[e]
        d = dst_ref[e]
        ds = pl.ds(d, 1)
        c = jnp.where(base + e >= half, c1, c0)
        m = a_ref[ds, :] + b_ref[pl.ds(s, 1), :] + c
        if op == 'sum':
            o1_ref[ds, :] = o1_ref[ds, :] + m
            o2_ref[ds, :] = o2_ref[ds, :] + m * m
        else:
            o1_ref[ds, :] = jnp.maximum(o1_ref[ds, :], m)
            o2_ref[ds, :] = jnp.minimum(o2_ref[ds, :], m)
        return carry

    jax.lax.fori_loop(0, src_ref.shape[0], body, 0)


def _pna_scatter(src, dst, A, B, c2, chunk=512):
    e = src.shape[0]
    n, h = A.shape
    nchunks = e // chunk
    out = jax.ShapeDtypeStruct((n, h), jnp.float32)
    nh_spec = pl.BlockSpec((n, h), lambda i: (0, 0))
    idx_spec = pl.BlockSpec((chunk,), lambda i: (i,),
                            memory_space=pltpu.SMEM)

    def run(op):
        return pl.pallas_call(
            functools.partial(_pna_scatter_kernel, chunk=chunk,
                              half=e // 2, op=op),
            grid=(nchunks,),
            in_specs=[
                idx_spec,
                idx_spec,
                nh_spec,
                nh_spec,
                pl.BlockSpec((2, h), lambda i: (0, 0)),
            ],
            out_specs=(nh_spec, nh_spec),
            out_shape=(out, out),
        )(src, dst, A, B, c2)

    s_sum, s_ssq = run('sum')
    s_max, s_min = run('minmax')
    return s_sum, s_max, s_min, s_ssq


def _post_kernel(h_ref, sum_ref, max_ref, min_ref, ssq_ref,
                 degcl_ref, amp_ref, att_ref,
                 wpost_ref, bpost_ref, wlin_ref, blin_ref, gam_ref, bet_ref,
                 o_ref):
    degc = degcl_ref[...]
    s_sum = sum_ref[...]
    mean = s_sum / degc
    mean2 = ssq_ref[...] / degc
    std = jnp.sqrt(jnp.maximum(mean2 - mean * mean, 0.0) + 1e-5)
    smax = max_ref[...]
    smax = jnp.where(jnp.isfinite(smax), smax, 0.0)
    smin = min_ref[...]
    smin = jnp.where(jnp.isfinite(smin), smin, 0.0)
    aggr = jnp.concatenate([mean, smax, smin, std, s_sum], axis=-1)
    full = jnp.concatenate(
        [h_ref[...], aggr, aggr * amp_ref[...], aggr * att_ref[...]], axis=-1
    )
    y = (
        jnp.dot(full, wpost_ref[...], preferred_element_type=jnp.float32)
        + bpost_ref[...]
    )
    y = (
        jnp.dot(y, wlin_ref[...], preferred_element_type=jnp.float32)
        + blin_ref[...]
    )
    y = gam_ref[...] * y * (1.0 / jnp.sqrt(jnp.float32(1.0 + 1e-5))) + bet_ref[...]
    o_ref[...] = jnp.maximum(y, 0.0)


def _post(h, s_sum, s_max, s_min, s_ssq, deg_cl, amp, att, lp, blk=1000):
    n, hd = h.shape
    k16 = lp['Wpost'].shape[0]
    blk_spec = pl.BlockSpec((blk, hd), lambda i: (i, 0))
    col_spec = pl.BlockSpec((blk, 1), lambda i: (i, 0))
    row_spec = pl.BlockSpec((1, hd), lambda i: (0, 0))
    return pl.pallas_call(
        _post_kernel,
        grid=(n // blk,),
        in_specs=[
            blk_spec, blk_spec, blk_spec, blk_spec, blk_spec,
            col_spec, col_spec, col_spec,
            pl.BlockSpec((k16, hd), lambda i: (0, 0)),
            row_spec,
            pl.BlockSpec((hd, hd), lambda i: (0, 0)),
            row_spec, row_spec, row_spec,
        ],
        out_specs=blk_spec,
        out_shape=jax.ShapeDtypeStruct((n, hd), jnp.float32),
    )(h, s_sum, s_max, s_min, s_ssq, deg_cl, amp, att,
      lp['Wpost'], lp['bpost'].reshape(1, hd), lp['Wlin'],
      lp['blin'].reshape(1, hd), lp['gamma'].reshape(1, hd),
      lp['beta'].reshape(1, hd))


def _pool_cls_kernel(h_ref, batch_ref, w1_ref, b1_ref, w2_ref, b2_ref,
                     w3_ref, b3_ref, o_ref, sum_ref, max_ref, cnt_ref,
                     *, nblk):
    i = pl.program_id(0)

    @pl.when(i == 0)
    def _():
        sum_ref[...] = jnp.zeros_like(sum_ref)
        max_ref[...] = jnp.full_like(max_ref, -jnp.inf)
        cnt_ref[...] = jnp.zeros_like(cnt_ref)

    h = h_ref[...]
    bvec = batch_ref[...]
    for g in range(_NUM_GRAPHS):
        is_g = bvec == g
        maskf = is_g.astype(jnp.float32)
        sum_ref[g:g + 1, :] = sum_ref[g:g + 1, :] + jnp.sum(
            h * maskf, axis=0, keepdims=True)
        max_ref[g:g + 1, :] = jnp.maximum(
            max_ref[g:g + 1, :],
            jnp.max(jnp.where(is_g, h, -jnp.inf), axis=0, keepdims=True))
        cnt_ref[g:g + 1, :] = cnt_ref[g:g + 1, :] + jnp.sum(maskf)

    @pl.when(i == nblk - 1)
    def _():
        psum = sum_ref[...]
        cnt = jnp.maximum(cnt_ref[:, 0:1], 1.0)
        pmax = max_ref[...]
        pmax = jnp.where(jnp.isfinite(pmax), pmax, 0.0)
        gmat = jnp.concatenate([psum / cnt, pmax, psum], axis=1)
        y = jnp.maximum(
            jnp.dot(gmat, w1_ref[...], preferred_element_type=jnp.float32)
            + b1_ref[...], 0.0)
        y = jnp.maximum(
            jnp.dot(y, w2_ref[...], preferred_element_type=jnp.float32)
            + b2_ref[...], 0.0)
        o_ref[...] = (
            jnp.dot(y, w3_ref[...], preferred_element_type=jnp.float32)
            + b3_ref[...]
        )


def _pool_cls(h, batch, cls, blk=1000):
    n, hd = h.shape
    k1 = cls['W1'].shape[0]
    h2 = cls['W2'].shape[1]
    out = cls['W3'].shape[1]
    nblk = n // blk
    return pl.pallas_call(
        functools.partial(_pool_cls_kernel, nblk=nblk),
        grid=(nblk,),
        in_specs=[
            pl.BlockSpec((blk, hd), lambda i: (i, 0)),
            pl.BlockSpec((blk, 1), lambda i: (i, 0)),
            pl.BlockSpec((k1, hd), lambda i: (0, 0)),
            pl.BlockSpec((1, hd), lambda i: (0, 0)),
            pl.BlockSpec((hd, h2), lambda i: (0, 0)),
            pl.BlockSpec((1, h2), lambda i: (0, 0)),
            pl.BlockSpec((h2, out), lambda i: (0, 0)),
            pl.BlockSpec((1, out), lambda i: (0, 0)),
        ],
        out_specs=pl.BlockSpec((_NUM_GRAPHS, out), lambda i: (0, 0)),
        out_shape=jax.ShapeDtypeStruct((_NUM_GRAPHS, out), jnp.float32),
        scratch_shapes=[
            pltpu.VMEM((_NUM_GRAPHS, hd), jnp.float32),
            pltpu.VMEM((_NUM_GRAPHS, hd), jnp.float32),
            pltpu.VMEM((_NUM_GRAPHS, 128), jnp.float32),
        ],
    )(h, batch.reshape(n, 1), cls['W1'], cls['b1'].reshape(1, hd),
      cls['W2'], cls['b2'].reshape(1, h2), cls['W3'], cls['b3'].reshape(1, out))


def kernel(x, edge_index, batch, params):
    n = x.shape[0]
    src, dst = edge_index[0], edge_index[1]
    p = params
    hd = p['proj']['W'].shape[1]

    h = _linear(x, p['proj']['W'], p['proj']['b'])

    deg = _degree(dst, n)
    deg_cl = jnp.clip(deg, 1.0, None)
    avg_log = jnp.mean(jnp.log(deg + 1.0))
    amp = jnp.log(deg_cl + 1.0) / avg_log
    att = avg_log / jnp.log(deg_cl + 1.0)

    zeros_b = jnp.zeros((hd,), jnp.float32)
    for lp in p['layers']:
        Wpre = lp['Wpre']
        A = _linear(h, Wpre[:hd], zeros_b)
        B = _linear(h, Wpre[hd:2 * hd], zeros_b)
        e0 = lp['be']
        e1 = lp['We'][1] + lp['be']
        c0 = e0 @ Wpre[2 * hd:] + lp['bpre']
        c1 = e1 @ Wpre[2 * hd:] + lp['bpre']
        c2 = jnp.stack([c0, c1])
        s_sum, s_max, s_min, s_ssq = _pna_scatter(src, dst, A, B, c2)
        h = _post(h, s_sum, s_max, s_min, s_ssq, deg_cl, amp, att, lp)

    return _pool_cls(h, batch, p['cls'])


# speedup vs baseline: 1.7235x; 1.7235x over previous
import functools

import jax
import jax.numpy as jnp
from jax.experimental import pallas as pl
from jax.experimental.pallas import tpu as pltpu

_NUM_GRAPHS = 16


def _linear_kernel(x_ref, w_ref, b_ref, o_ref):
    o_ref[...] = (
        jnp.dot(x_ref[...], w_ref[...], preferred_element_type=jnp.float32)
        + b_ref[...]
    )


def _linear(x, W, b, blk=1000):
    n, k = x.shape
    m = W.shape[1]
    return pl.pallas_call(
        _linear_kernel,
        grid=(n // blk,),
        in_specs=[
            pl.BlockSpec((blk, k), lambda i: (i, 0)),
            pl.BlockSpec((k, m), lambda i: (0, 0)),
            pl.BlockSpec((1, m), lambda i: (0, 0)),
        ],
        out_specs=pl.BlockSpec((blk, m), lambda i: (i, 0)),
        out_shape=jax.ShapeDtypeStruct((n, m), jnp.float32),
    )(x, W, b.reshape(1, m))


def _degree_kernel(dst_ref, deg_ref):
    @pl.when(pl.program_id(0) == 0)
    def _():
        deg_ref[...] = jnp.zeros_like(deg_ref)

    def body(e, carry):
        d = dst_ref[e]
        deg_ref[pl.ds(d, 1), :] = deg_ref[pl.ds(d, 1), :] + 1.0
        return carry

    jax.lax.fori_loop(0, dst_ref.shape[0], body, 0)


def _degree(dst, n, chunk=512):
    e = dst.shape[0]
    return pl.pallas_call(
        _degree_kernel,
        grid=(e // chunk,),
        in_specs=[
            pl.BlockSpec((chunk,), lambda i: (i,), memory_space=pltpu.SMEM)
        ],
        out_specs=pl.BlockSpec((n, 1), lambda i: (0, 0)),
        out_shape=jax.ShapeDtypeStruct((n, 1), jnp.float32),
    )(dst)


def _pna_scatter_kernel(src_ref, dst_ref, a_ref, b_ref, c_ref,
                        sum_ref, ssq_ref, max_ref, min_ref, *, chunk, half):
    i = pl.program_id(0)

    @pl.when(i == 0)
    def _():
        sum_ref[...] = jnp.zeros_like(sum_ref)
        ssq_ref[...] = jnp.zeros_like(ssq_ref)
        max_ref[...] = jnp.full_like(max_ref, -jnp.inf)
        min_ref[...] = jnp.full_like(min_ref, jnp.inf)

    c0 = c_ref[0:1, :]
    c1 = c_ref[1:2, :]
    base = i * chunk

    def body(e, carry):
        s = src_ref[e]
        d = dst_ref[e]
        ds = pl.ds(d, 1)
        c = jnp.where(base + e >= half, c1, c0)
        m = a_ref[ds, :] + b_ref[pl.ds(s, 1), :] + c
        sum_ref[ds, :] = sum_ref[ds, :] + m
        ssq_ref[ds, :] = ssq_ref[ds, :] + m * m
        max_ref[ds, :] = jnp.maximum(max_ref[ds, :], m)
        min_ref[ds, :] = jnp.minimum(min_ref[ds, :], m)
        return carry

    jax.lax.fori_loop(0, src_ref.shape[0], body, 0)


def _pna_scatter(src, dst, A, B, c2, chunk=512):
    e = src.shape[0]
    n, h = A.shape
    nchunks = e // chunk
    out = jax.ShapeDtypeStruct((n, h), jnp.float32)
    nh_spec = pl.BlockSpec((n, h), lambda i: (0, 0))
    idx_spec = pl.BlockSpec((chunk,), lambda i: (i,),
                            memory_space=pltpu.SMEM)
    s_sum, s_ssq, s_max, s_min = pl.pallas_call(
        functools.partial(_pna_scatter_kernel, chunk=chunk, half=e // 2),
        grid=(nchunks,),
        in_specs=[
            idx_spec,
            idx_spec,
            nh_spec,
            nh_spec,
            pl.BlockSpec((2, h), lambda i: (0, 0)),
        ],
        out_specs=(nh_spec, nh_spec, nh_spec, nh_spec),
        out_shape=(out, out, out, out),
    )(src, dst, A, B, c2)
    return s_sum, s_max, s_min, s_ssq


def _post_kernel(h_ref, sum_ref, max_ref, min_ref, ssq_ref,
                 degcl_ref, amp_ref, att_ref,
                 wpost_ref, bpost_ref, wlin_ref, blin_ref, gam_ref, bet_ref,
                 o_ref):
    degc = degcl_ref[...]
    s_sum = sum_ref[...]
    mean = s_sum / degc
    mean2 = ssq_ref[...] / degc
    std = jnp.sqrt(jnp.maximum(mean2 - mean * mean, 0.0) + 1e-5)
    smax = max_ref[...]
    smax = jnp.where(jnp.isfinite(smax), smax, 0.0)
    smin = min_ref[...]
    smin = jnp.where(jnp.isfinite(smin), smin, 0.0)
    aggr = jnp.concatenate([mean, smax, smin, std, s_sum], axis=-1)
    full = jnp.concatenate(
        [h_ref[...], aggr, aggr * amp_ref[...], aggr * att_ref[...]], axis=-1
    )
    y = (
        jnp.dot(full, wpost_ref[...], preferred_element_type=jnp.float32)
        + bpost_ref[...]
    )
    y = (
        jnp.dot(y, wlin_ref[...], preferred_element_type=jnp.float32)
        + blin_ref[...]
    )
    y = gam_ref[...] * y * (1.0 / jnp.sqrt(jnp.float32(1.0 + 1e-5))) + bet_ref[...]
    o_ref[...] = jnp.maximum(y, 0.0)


def _post(h, s_sum, s_max, s_min, s_ssq, deg_cl, amp, att, lp, blk=1000):
    n, hd = h.shape
    k16 = lp['Wpost'].shape[0]
    blk_spec = pl.BlockSpec((blk, hd), lambda i: (i, 0))
    col_spec = pl.BlockSpec((blk, 1), lambda i: (i, 0))
    row_spec = pl.BlockSpec((1, hd), lambda i: (0, 0))
    return pl.pallas_call(
        _post_kernel,
        grid=(n // blk,),
        in_specs=[
            blk_spec, blk_spec, blk_spec, blk_spec, blk_spec,
            col_spec, col_spec, col_spec,
            pl.BlockSpec((k16, hd), lambda i: (0, 0)),
            row_spec,
            pl.BlockSpec((hd, hd), lambda i: (0, 0)),
            row_spec, row_spec, row_spec,
        ],
        out_specs=blk_spec,
        out_shape=jax.ShapeDtypeStruct((n, hd), jnp.float32),
    )(h, s_sum, s_max, s_min, s_ssq, deg_cl, amp, att,
      lp['Wpost'], lp['bpost'].reshape(1, hd), lp['Wlin'],
      lp['blin'].reshape(1, hd), lp['gamma'].reshape(1, hd),
      lp['beta'].reshape(1, hd))


def _pool_cls_kernel(h_ref, batch_ref, w1_ref, b1_ref, w2_ref, b2_ref,
                     w3_ref, b3_ref, o_ref, sum_ref, max_ref, cnt_ref,
                     *, nblk):
    i = pl.program_id(0)

    @pl.when(i == 0)
    def _():
        sum_ref[...] = jnp.zeros_like(sum_ref)
        max_ref[...] = jnp.full_like(max_ref, -jnp.inf)
        cnt_ref[...] = jnp.zeros_like(cnt_ref)

    h = h_ref[...]
    bvec = batch_ref[...]
    for g in range(_NUM_GRAPHS):
        is_g = bvec == g
        maskf = is_g.astype(jnp.float32)
        sum_ref[g:g + 1, :] = sum_ref[g:g + 1, :] + jnp.sum(
            h * maskf, axis=0, keepdims=True)
        max_ref[g:g + 1, :] = jnp.maximum(
            max_ref[g:g + 1, :],
            jnp.max(jnp.where(is_g, h, -jnp.inf), axis=0, keepdims=True))
        cnt_ref[g:g + 1, :] = cnt_ref[g:g + 1, :] + jnp.sum(maskf)

    @pl.when(i == nblk - 1)
    def _():
        psum = sum_ref[...]
        cnt = jnp.maximum(cnt_ref[:, 0:1], 1.0)
        pmax = max_ref[...]
        pmax = jnp.where(jnp.isfinite(pmax), pmax, 0.0)
        gmat = jnp.concatenate([psum / cnt, pmax, psum], axis=1)
        y = jnp.maximum(
            jnp.dot(gmat, w1_ref[...], preferred_element_type=jnp.float32)
            + b1_ref[...], 0.0)
        y = jnp.maximum(
            jnp.dot(y, w2_ref[...], preferred_element_type=jnp.float32)
            + b2_ref[...], 0.0)
        o_ref[...] = (
            jnp.dot(y, w3_ref[...], preferred_element_type=jnp.float32)
            + b3_ref[...]
        )


def _pool_cls(h, batch, cls, blk=1000):
    n, hd = h.shape
    k1 = cls['W1'].shape[0]
    h2 = cls['W2'].shape[1]
    out = cls['W3'].shape[1]
    nblk = n // blk
    return pl.pallas_call(
        functools.partial(_pool_cls_kernel, nblk=nblk),
        grid=(nblk,),
        in_specs=[
            pl.BlockSpec((blk, hd), lambda i: (i, 0)),
            pl.BlockSpec((blk, 1), lambda i: (i, 0)),
            pl.BlockSpec((k1, hd), lambda i: (0, 0)),
            pl.BlockSpec((1, hd), lambda i: (0, 0)),
            pl.BlockSpec((hd, h2), lambda i: (0, 0)),
            pl.BlockSpec((1, h2), lambda i: (0, 0)),
            pl.BlockSpec((h2, out), lambda i: (0, 0)),
            pl.BlockSpec((1, out), lambda i: (0, 0)),
        ],
        out_specs=pl.BlockSpec((_NUM_GRAPHS, out), lambda i: (0, 0)),
        out_shape=jax.ShapeDtypeStruct((_NUM_GRAPHS, out), jnp.float32),
        scratch_shapes=[
            pltpu.VMEM((_NUM_GRAPHS, hd), jnp.float32),
            pltpu.VMEM((_NUM_GRAPHS, hd), jnp.float32),
            pltpu.VMEM((_NUM_GRAPHS, 128), jnp.float32),
        ],
    )(h, batch.reshape(n, 1), cls['W1'], cls['b1'].reshape(1, hd),
      cls['W2'], cls['b2'].reshape(1, h2), cls['W3'], cls['b3'].reshape(1, out))


def kernel(x, edge_index, batch, params):
    n = x.shape[0]
    src, dst = edge_index[0], edge_index[1]
    p = params
    hd = p['proj']['W'].shape[1]

    h = _linear(x, p['proj']['W'], p['proj']['b'])

    deg = _degree(dst, n)
    deg_cl = jnp.clip(deg, 1.0, None)
    avg_log = jnp.mean(jnp.log(deg + 1.0))
    amp = jnp.log(deg_cl + 1.0) / avg_log
    att = avg_log / jnp.log(deg_cl + 1.0)

    zeros_b = jnp.zeros((hd,), jnp.float32)
    for lp in p['layers']:
        Wpre = lp['Wpre']
        A = _linear(h, Wpre[:hd], zeros_b)
        B = _linear(h, Wpre[hd:2 * hd], zeros_b)
        e0 = lp['be']
        e1 = lp['We'][1] + lp['be']
        c0 = e0 @ Wpre[2 * hd:] + lp['bpre']
        c1 = e1 @ Wpre[2 * hd:] + lp['bpre']
        c2 = jnp.stack([c0, c1])
        s_sum, s_max, s_min, s_ssq = _pna_scatter(src, dst, A, B, c2)
        h = _post(h, s_sum, s_max, s_min, s_ssq, deg_cl, amp, att, lp)

    return _pool_cls(h, batch, p['cls'])
